# baseline (speedup 1.0000x reference)
import functools

import jax
import jax.numpy as jnp
from jax import lax
from jax.experimental import pallas as pl
from jax.experimental.pallas import tpu as pltpu
from jax.experimental.pallas import tpu_sc as plsc

_N_TOKENS = 10
_B = 1024
_L = 210
_D = 64
_CONTENT = _L - _N_TOKENS
_NUM_WORKERS = 32
_BATCH_PER_WORKER = _B // _NUM_WORKERS


def _soft_embedding_sc(idx, wte, learned_embedding):
    mesh = plsc.VectorSubcoreMesh(core_axis_name="c", subcore_axis_name="s")

    @functools.partial(
        pl.kernel,
        mesh=mesh,
        out_type=jax.ShapeDtypeStruct((_B, _L, _D), jnp.float32),
        scratch_types=[
            pltpu.VMEM((_CONTENT,), jnp.int32),
            pltpu.VMEM((_L, _D), jnp.float32),
            pltpu.SemaphoreType.DMA,
        ],
        compiler_params=pltpu.CompilerParams(use_tc_tiling_on_sc=False),
    )
    def k(idx_hbm, wte_hbm, le_hbm, out_hbm, idx_v, buf_v, sem):
        wid = lax.axis_index("s") * 2 + lax.axis_index("c")
        base = wid * _BATCH_PER_WORKER
        pltpu.sync_copy(le_hbm, buf_v.at[pl.ds(0, _N_TOKENS)])

        def body(i, carry):
            b = base + i
            pltpu.sync_copy(idx_hbm.at[b], idx_v)
            pltpu.async_copy(
                wte_hbm.at[idx_v], buf_v.at[pl.ds(_N_TOKENS, _CONTENT)], sem
            ).wait()
            pltpu.sync_copy(buf_v, out_hbm.at[b])
            return carry

        lax.fori_loop(0, _BATCH_PER_WORKER, body, 0)

    return k(idx, wte, learned_embedding)


def kernel(tokens, wte, learned_embedding):
    idx = tokens[:, _N_TOKENS:]
    return _soft_embedding_sc(idx, wte, learned_embedding)


# speedup vs baseline: 1.0361x; 1.0361x over previous
import functools

import jax
import jax.numpy as jnp
from jax import lax
from jax.experimental import pallas as pl
from jax.experimental.pallas import tpu as pltpu
from jax.experimental.pallas import tpu_sc as plsc

_N_TOKENS = 10
_B = 1024
_L = 210
_D = 64
_CONTENT = _L - _N_TOKENS
_NUM_WORKERS = 32
_BPW = _B // _NUM_WORKERS
_G = 4
_NCHUNK = _BPW // _G
_NBUF = 2


def _soft_embedding_sc(idx, wte, learned_embedding):
    mesh = plsc.VectorSubcoreMesh(core_axis_name="c", subcore_axis_name="s")

    @functools.partial(
        pl.kernel,
        mesh=mesh,
        out_type=jax.ShapeDtypeStruct((_B, _L, _D), jnp.float32),
        scratch_types=[
            pltpu.VMEM((_BPW, _CONTENT), jnp.int32),
            pltpu.VMEM((_NBUF, _G, _L, _D), jnp.float32),
            pltpu.SemaphoreType.DMA,
            pltpu.SemaphoreType.DMA,
            pltpu.SemaphoreType.DMA,
            pltpu.SemaphoreType.DMA,
        ],
        compiler_params=pltpu.CompilerParams(use_tc_tiling_on_sc=False),
    )
    def k(idx_hbm, wte_hbm, le_hbm, out_hbm, idx_v, bufs, sg0, sg1, sw0, sw1):
        wid = lax.axis_index("s") * 2 + lax.axis_index("c")
        base = wid * _BPW
        sem_g = (sg0, sg1)
        sem_w = (sw0, sw1)

        pltpu.sync_copy(idx_hbm.at[pl.ds(base, _BPW)], idx_v)
        for p in range(_NBUF):
            for b in range(_G):
                pltpu.sync_copy(le_hbm, bufs.at[p, b, pl.ds(0, _N_TOKENS)])

        def gather_descs(g, p, issue):
            for b in range(_G):
                src = wte_hbm.at[idx_v.at[g * _G + b]]
                dst = bufs.at[p, b, pl.ds(_N_TOKENS, _CONTENT)]
                if issue:
                    pltpu.async_copy(src, dst, sem_g[p])
                else:
                    pltpu.make_async_copy(src, dst, sem_g[p]).wait()

        def write_desc(g, p, issue):
            src = bufs.at[p]
            dst = out_hbm.at[pl.ds(base + g * _G, _G)]
            if issue:
                pltpu.async_copy(src, dst, sem_w[p])
            else:
                pltpu.make_async_copy(src, dst, sem_w[p]).wait()

        gather_descs(0, 0, True)
        gather_descs(1, 1, True)

        def body(g2, carry):
            for p in range(_NBUF):
                g = g2 * _NBUF + p
                gather_descs(g, p, False)
                write_desc(g, p, True)

            @pl.when(g2 < _NCHUNK // _NBUF - 1)
            def _():
                for p in range(_NBUF):
                    g = g2 * _NBUF + p
                    write_desc(g, p, False)
                    gather_descs(g + _NBUF, p, True)

            return carry

        lax.fori_loop(0, _NCHUNK // _NBUF, body, 0)
        write_desc(_NCHUNK - 2, 0, False)
        write_desc(_NCHUNK - 1, 1, False)

    return k(idx, wte, learned_embedding)


def kernel(tokens, wte, learned_embedding):
    idx = tokens[:, _N_TOKENS:]
    return _soft_embedding_sc(idx, wte, learned_embedding)


# speedup vs baseline: 1.1282x; 1.0889x over previous
import functools

import jax
import jax.numpy as jnp
from jax import lax
from jax.experimental import pallas as pl
from jax.experimental.pallas import tpu as pltpu
from jax.experimental.pallas import tpu_sc as plsc

_N_TOKENS = 10
_B = 1024
_L = 210
_D = 64
_CONTENT = _L - _N_TOKENS
_NUM_WORKERS = 32
_BPW = _B // _NUM_WORKERS
_G = 4
_NCHUNK = _BPW // _G
_NBUF = 2
_VOCAB2 = 2_000_000


def _soft_embedding_sc(idx, wte2, learned_embedding):
    mesh = plsc.VectorSubcoreMesh(core_axis_name="c", subcore_axis_name="s")

    @functools.partial(
        pl.kernel,
        mesh=mesh,
        out_type=jax.ShapeDtypeStruct((_B, _L, _D), jnp.float32),
        scratch_types=[
            pltpu.VMEM((_BPW, _CONTENT), jnp.int32),
            pltpu.VMEM((_NBUF, _G, _L, _D), jnp.float32),
            pltpu.SemaphoreType.DMA,
            pltpu.SemaphoreType.DMA,
            pltpu.SemaphoreType.DMA,
            pltpu.SemaphoreType.DMA,
        ],
        compiler_params=pltpu.CompilerParams(use_tc_tiling_on_sc=False),
    )
    def k(idx_hbm, wte_hbm, le_hbm, out_hbm, idx_v, bufs, sg0, sg1, sw0, sw1):
        wid = lax.axis_index("s") * 2 + lax.axis_index("c")
        base = wid * _BPW
        sem_g = (sg0, sg1)
        sem_w = (sw0, sw1)

        pltpu.sync_copy(idx_hbm.at[pl.ds(base, _BPW)], idx_v)
        for p in range(_NBUF):
            for b in range(_G):
                pltpu.sync_copy(le_hbm, bufs.at[p, b, pl.ds(0, _N_TOKENS)])

        def gather_descs(g, p, issue):
            for b in range(_G):
                src = wte_hbm.at[idx_v.at[g * _G + b]]
                dst = bufs.at[p, b, pl.ds(_N_TOKENS, _CONTENT)]
                if issue:
                    pltpu.async_copy(src, dst, sem_g[p])
                else:
                    pltpu.make_async_copy(src, dst, sem_g[p]).wait()

        def write_desc(g, p, issue):
            src = bufs.at[p]
            dst = out_hbm.at[pl.ds(base + g * _G, _G)]
            if issue:
                pltpu.async_copy(src, dst, sem_w[p])
            else:
                pltpu.make_async_copy(src, dst, sem_w[p]).wait()

        gather_descs(0, 0, True)
        gather_descs(1, 1, True)

        def body(g2, carry):
            for p in range(_NBUF):
                g = g2 * _NBUF + p
                gather_descs(g, p, False)
                write_desc(g, p, True)

            @pl.when(g2 < _NCHUNK // _NBUF - 1)
            def _():
                for p in range(_NBUF):
                    g = g2 * _NBUF + p
                    write_desc(g, p, False)
                    gather_descs(g + _NBUF, p, True)

            return carry

        lax.fori_loop(0, _NCHUNK // _NBUF, body, 0)
        write_desc(_NCHUNK - 2, 0, False)
        write_desc(_NCHUNK - 1, 1, False)

    return k(idx, wte2, learned_embedding)


def kernel(tokens, wte, learned_embedding):
    wte2 = jnp.pad(wte, ((0, 0), (0, _D))).reshape(_VOCAB2, _D)
    idx = tokens[:, _N_TOKENS:] * 2
    return _soft_embedding_sc(idx, wte2, learned_embedding)
